# baseline (speedup 1.0000x reference)
import jax
import jax.numpy as jnp
from jax import lax
from jax.experimental import pallas as pl
from jax.experimental.pallas import tpu as pltpu
from jax.experimental.pallas import tpu_sc as plsc

NF = 8
D = 64
V = 35
NP = NF // 2
D2 = 2 * D
VP = V * V
DW = NF * D
CH = 64
RPC = CH * NP
GL = 128
NG = RPC // GL
NWORK = 32


def _body(xp_hbm, wp_hbm, out_hbm, idx_v, tbl_sh, st0, st1,
          gsem0, gsem1, wsem0, wsem1):
    nrow = idx_v.shape[0]
    nch = nrow // NG
    wid = lax.axis_index("s") * 2 + lax.axis_index("c")

    @pl.when(lax.axis_index("s") == 0)
    def _():
        pltpu.sync_copy(wp_hbm, tbl_sh)
    pltpu.sync_copy(xp_hbm.at[wid], idx_v)
    plsc.subcore_barrier()

    stages = (st0, st1)
    gsems = (gsem0, gsem1)
    wsems = (wsem0, wsem1)

    def step(c, slot):
        stage, gsem, wsem = stages[slot], gsems[slot], wsems[slot]

        @pl.when(c >= 2)
        def _():
            pltpu.make_async_copy(
                stage, out_hbm.at[wid, pl.ds(0, RPC)], wsem).wait()

        for q in range(NG):
            pltpu.async_copy(
                tbl_sh.at[idx_v.at[c * NG + q]],
                stage.at[pl.ds(q * GL, GL)], gsem)
        for q in range(NG):
            pltpu.make_async_copy(
                tbl_sh.at[idx_v.at[0]],
                stage.at[pl.ds(q * GL, GL)], gsem).wait()

        pltpu.async_copy(
            stage, out_hbm.at[wid, pl.ds(c * RPC, RPC)], wsem)

    def pair(o, _):
        for phase in range(2):
            step(2 * o + phase, phase)
        return 0
    lax.fori_loop(0, nch // 2, pair, 0)

    for slot in range(2):
        pltpu.make_async_copy(
            stages[slot], out_hbm.at[wid, pl.ds(0, RPC)], wsems[slot]).wait()


def kernel(x, W0, W1, W2, W3, W4, W5, W6, W7):
    B, nf, L = x.shape
    assert nf == NF and B == NWORK and (L * NP) % (2 * NG * GL) == 0
    tables = (W0, W1, W2, W3, W4, W5, W6, W7)
    pairs = []
    for p in range(NP):
        a = jnp.broadcast_to(tables[2 * p][:V, None, :], (V, V, D))
        b = jnp.broadcast_to(tables[2 * p + 1][None, :V, :], (V, V, D))
        pairs.append(jnp.concatenate([a, b], axis=-1).reshape(VP, D2))
    wpair = jnp.concatenate(pairs, axis=0)
    xi = x.astype(jnp.int32)
    xp = (xi[:, 0::2] * V + xi[:, 1::2]
          + (VP * jnp.arange(NP, dtype=jnp.int32))[None, :, None])
    xp = xp.transpose(0, 2, 1).reshape(B, (L * NP) // GL, GL)

    mesh = plsc.VectorSubcoreMesh(core_axis_name="c", subcore_axis_name="s")
    f = pl.kernel(
        _body,
        compiler_params=pltpu.CompilerParams(
            use_tc_tiling_on_sc=False, needs_layout_passes=False),
        out_type=jax.ShapeDtypeStruct((B, L * NP, D2), jnp.float32),
        mesh=mesh,
        scratch_types=[
            pltpu.VMEM(((L * NP) // GL, GL), jnp.int32),
            pltpu.VMEM_SHARED((NP * VP, D2), jnp.float32),
            pltpu.VMEM((RPC, D2), jnp.float32),
            pltpu.VMEM((RPC, D2), jnp.float32),
            pltpu.SemaphoreType.DMA,
            pltpu.SemaphoreType.DMA,
            pltpu.SemaphoreType.DMA,
            pltpu.SemaphoreType.DMA,
        ],
    )
    out = f(xp, wpair)
    return out.reshape(B, L, DW)
